# baseline (speedup 1.0000x reference)
import functools

import jax
import jax.numpy as jnp
from jax import lax
from jax.experimental import pallas as pl
from jax.experimental.pallas import tpu as pltpu
from jax.experimental.pallas import tpu_sc as plsc

_B, _F, _V, _D = 16384, 26, 1000000, 32
_NC, _NS, _L = 2, 16, 16
_NW = _NC * _NS


@functools.lru_cache(maxsize=None)
def _build_relayout(V, D):
    NCOL = (V + 127) // 128
    NFULL = V // 128
    MAXCOL = -(-NCOL // _NW)
    L = _L

    mesh = plsc.VectorSubcoreMesh(
        core_axis_name="c", subcore_axis_name="s", num_cores=_NC,
        num_subcores=_NS)

    @functools.partial(
        pl.kernel,
        out_type=jax.ShapeDtypeStruct((V * D,), jnp.float32),
        mesh=mesh,
        scratch_types=[
            pltpu.VMEM((D, 128), jnp.float32),
            pltpu.VMEM((128 * D,), jnp.float32),
            pltpu.SemaphoreType.DMA,
        ],
        compiler_params=pltpu.CompilerParams(
            needs_layout_passes=False, use_tc_tiling_on_sc=True),
    )
    def relayout(em1t, out, tin, tout, sem):
        wid = lax.axis_index("s") * _NC + lax.axis_index("c")
        d_iota = lax.iota(jnp.int32, L)

        def transpose_col(width):
            for vv in range(width):
                for h in range(D // L):
                    vals = plsc.load_gather(
                        tin, [d_iota + h * L, jnp.full((L,), vv, jnp.int32)])
                    tout[pl.ds(vv * D + h * L, L)] = vals

        def col(i, carry):
            c = wid + i * _NW

            @pl.when(c < NFULL)
            def _full():
                for g in range(D // 8):
                    pltpu.sync_copy(
                        em1t.at[pl.ds(8 * g, 8), pl.ds(c * 128, 128)],
                        tin.at[pl.ds(8 * g, 8), :])
                transpose_col(128)
                pltpu.sync_copy(tout, out.at[pl.ds(c * 128 * D, 128 * D)])

            @pl.when(jnp.logical_and(c >= NFULL, c < NCOL))
            def _tail():
                width = V - NFULL * 128
                for g in range(D // 8):
                    pltpu.sync_copy(
                        em1t.at[pl.ds(8 * g, 8), pl.ds(c * 128, width)],
                        tin.at[pl.ds(8 * g, 8), pl.ds(0, width)])
                transpose_col(width)
                pltpu.sync_copy(tout.at[pl.ds(0, width * D)],
                                out.at[pl.ds(c * 128 * D, width * D)])

            return carry

        lax.fori_loop(0, MAXCOL, col, 0, unroll=False)

    return relayout


@functools.lru_cache(maxsize=None)
def _build_fm(B, F, V, D, R):
    NW = _NW
    RW = B // NW
    NCHUNK = RW // R
    NIDX = R * F
    SLEN = 104 if NIDX % 104 == 0 else 8
    while NIDX % SLEN or SLEN > 128:
        SLEN -= 8
    NSTREAM = NIDX // SLEN
    L = _L

    mesh = plsc.VectorSubcoreMesh(
        core_axis_name="c", subcore_axis_name="s", num_cores=_NC,
        num_subcores=_NS)

    @functools.partial(
        pl.kernel,
        out_type=(
            jax.ShapeDtypeStruct((B * F,), jnp.float32),
            jax.ShapeDtypeStruct((B * D,), jnp.float32),
        ),
        mesh=mesh,
        scratch_types=[
            pltpu.VMEM((NIDX,), jnp.int32),
            pltpu.VMEM((NIDX,), jnp.float32),
            pltpu.VMEM((NIDX,), jnp.float32),
            pltpu.VMEM((NIDX, D), jnp.float32),
            pltpu.VMEM((NIDX,), jnp.float32),
            pltpu.VMEM((R * D,), jnp.float32),
            pltpu.SemaphoreType.DMA,
        ],
        compiler_params=pltpu.CompilerParams(
            needs_layout_passes=False, use_tc_tiling_on_sc=False),
    )
    def fm(fi, fv, em1, em2, y1, y2, idx_v, fv_v, w2_v, rows_v, y1_v, y2_v,
           sem):
        wid = lax.axis_index("s") * _NC + lax.axis_index("c")

        def chunk(c, carry):
            row0 = wid * RW + c * R
            flat0 = row0 * F
            pltpu.sync_copy(fi.at[pl.ds(flat0, NIDX)], idx_v)
            pltpu.sync_copy(fv.at[pl.ds(flat0, NIDX)], fv_v)
            handles = []
            for k in range(NSTREAM):
                sl = pl.ds(k * SLEN, SLEN)
                handles.append(
                    pltpu.async_copy(em1.at[idx_v.at[sl]], rows_v.at[sl], sem))
                handles.append(
                    pltpu.async_copy(em2.at[idx_v.at[sl]], w2_v.at[sl], sem))
            for h in handles:
                h.wait()

            for i in range(NIDX // L):
                s = pl.ds(i * L, L)
                y1_v[s] = w2_v[s] * fv_v[s]

            def row(b, _):
                fb = b * F
                z = jnp.zeros((L,), jnp.float32)
                a_lo, a_hi, s_lo, s_hi = z, z, z, z
                for f in range(F):
                    r = fb + f
                    fvf = plsc.load_gather(fv_v, [jnp.full((L,), r, jnp.int32)])
                    lo = rows_v[r, pl.ds(0, L)] * fvf
                    hi = rows_v[r, pl.ds(L, L)] * fvf
                    a_lo = a_lo + lo
                    a_hi = a_hi + hi
                    s_lo = s_lo + lo * lo
                    s_hi = s_hi + hi * hi
                y2_v[pl.ds(b * D, L)] = 0.5 * (a_lo * a_lo - s_lo)
                y2_v[pl.ds(b * D + L, L)] = 0.5 * (a_hi * a_hi - s_hi)
                return 0

            lax.fori_loop(0, R, row, 0, unroll=False)

            pltpu.sync_copy(y1_v, y1.at[pl.ds(flat0, NIDX)])
            pltpu.sync_copy(y2_v, y2.at[pl.ds(row0 * D, R * D)])
            return carry

        lax.fori_loop(0, NCHUNK, chunk, 0, unroll=False)

    return fm


def kernel(feat_index, feat_value, em1_table, em2_table):
    B, F = feat_index.shape
    V, D = em1_table.shape
    fi = feat_index.astype(jnp.int32).reshape(B * F)
    fv = feat_value.reshape(B * F)
    em2 = em2_table.reshape(V)
    em1_lin = _build_relayout(V, D)(em1_table.T).reshape(V, D)
    y1, y2 = _build_fm(B, F, V, D, 32)(fi, fv, em1_lin, em2)
    return y1.reshape(B, F), y2.reshape(B, D)


# speedup vs baseline: 2.0114x; 2.0114x over previous
import functools

import jax
import jax.numpy as jnp
from jax import lax
from jax.experimental import pallas as pl
from jax.experimental.pallas import tpu as pltpu
from jax.experimental.pallas import tpu_sc as plsc

_B, _F, _V, _D = 16384, 26, 1000000, 32
_NC, _NS, _L = 2, 16, 16
_NW = _NC * _NS


@functools.lru_cache(maxsize=None)
def _build_relayout(V, D):
    NCOL = (V + 127) // 128
    NFULL = V // 128
    TAILW = V - NFULL * 128
    CPW = -(-NFULL // _NW)
    NPAIR = -(-CPW // 2)
    CB = 128 * D
    L = _L

    mesh = plsc.VectorSubcoreMesh(
        core_axis_name="c", subcore_axis_name="s", num_cores=_NC,
        num_subcores=_NS)

    @functools.partial(
        pl.kernel,
        out_type=jax.ShapeDtypeStruct((NCOL * CB,), jnp.float32),
        mesh=mesh,
        scratch_types=[
            pltpu.VMEM((D, 128), jnp.float32),
            pltpu.VMEM((D, 128), jnp.float32),
            pltpu.VMEM((CB,), jnp.float32),
            pltpu.VMEM((CB,), jnp.float32),
            pltpu.SemaphoreType.DMA,
            pltpu.SemaphoreType.DMA,
            pltpu.SemaphoreType.DMA,
            pltpu.SemaphoreType.DMA,
        ],
        compiler_params=pltpu.CompilerParams(
            needs_layout_passes=False, use_tc_tiling_on_sc=True),
    )
    def relayout(em1t, out, tin_a, tin_b, tout_a, tout_b, isem_a, isem_b,
                 osem_a, osem_b):
        wid = lax.axis_index("s") * _NC + lax.axis_index("c")
        c0 = wid * CPW
        ncols = jnp.minimum(NFULL - c0, CPW)
        i32 = lax.iota(jnp.int32, L) * D

        def fire_in(tin, sem, c):
            pltpu.async_copy(em1t.at[:, pl.ds(c * 128, 128)], tin, sem)

        def wait_in(tin, sem):
            pltpu.make_async_copy(em1t.at[:, pl.ds(0, 128)], tin, sem).wait()

        def fire_out(tout, sem, c):
            pltpu.async_copy(tout, out.at[pl.ds(c * CB, CB)], sem)

        def wait_out(tout, sem):
            pltpu.make_async_copy(tout, out.at[pl.ds(0, CB)], sem).wait()

        def transpose(tin, tout, ngrp):
            for d in range(D):
                for grp in range(ngrp):
                    vals = tin[d, pl.ds(grp * L, L)]
                    plsc.store_scatter(tout, [i32 + (grp * L * D + d)], vals)

        fire_in(tin_a, isem_a, c0)

        def pair(i, carry):
            ca = c0 + 2 * i
            cb = ca + 1
            cn = ca + 2
            va = 2 * i < ncols
            vb = 2 * i + 1 < ncols
            vn = 2 * i + 2 < ncols

            @pl.when(va)
            def _wa():
                wait_in(tin_a, isem_a)

            @pl.when(vb)
            def _fb():
                fire_in(tin_b, isem_b, cb)

            @pl.when(jnp.logical_and(va, i > 0))
            def _woa():
                wait_out(tout_a, osem_a)

            @pl.when(va)
            def _ta():
                transpose(tin_a, tout_a, 8)
                fire_out(tout_a, osem_a, ca)

            @pl.when(vb)
            def _wb():
                wait_in(tin_b, isem_b)

            @pl.when(vn)
            def _fn():
                fire_in(tin_a, isem_a, cn)

            @pl.when(jnp.logical_and(vb, i > 0))
            def _wob():
                wait_out(tout_b, osem_b)

            @pl.when(vb)
            def _tb():
                transpose(tin_b, tout_b, 8)
                fire_out(tout_b, osem_b, cb)

            return carry

        lax.fori_loop(0, NPAIR, pair, 0, unroll=False)
        wait_out(tout_a, osem_a)
        wait_out(tout_b, osem_b)

        if TAILW:
            @pl.when(wid == 0)
            def _tail():
                fire_in(tin_a, isem_a, NFULL + wid)
                wait_in(tin_a, isem_a)
                transpose(tin_a, tout_a, -(-TAILW // L))
                pltpu.sync_copy(tout_a.at[pl.ds(0, TAILW * D)],
                                out.at[pl.ds(NFULL * CB, TAILW * D)])

    return relayout


@functools.lru_cache(maxsize=None)
def _build_fm(B, F, V, D, R):
    NW = _NW
    RW = B // NW
    NCHUNK = RW // R
    NIDX = R * F
    SLEN = 104 if NIDX % 104 == 0 else 8
    while NIDX % SLEN or SLEN > 128:
        SLEN -= 8
    NSTREAM = NIDX // SLEN
    L = _L

    mesh = plsc.VectorSubcoreMesh(
        core_axis_name="c", subcore_axis_name="s", num_cores=_NC,
        num_subcores=_NS)

    @functools.partial(
        pl.kernel,
        out_type=(
            jax.ShapeDtypeStruct((B * F,), jnp.float32),
            jax.ShapeDtypeStruct((B * D,), jnp.float32),
        ),
        mesh=mesh,
        scratch_types=[
            pltpu.VMEM((NIDX,), jnp.int32),
            pltpu.VMEM((NIDX,), jnp.float32),
            pltpu.VMEM((NIDX,), jnp.float32),
            pltpu.VMEM((NIDX, D), jnp.float32),
            pltpu.VMEM((NIDX,), jnp.float32),
            pltpu.VMEM((R * D,), jnp.float32),
            pltpu.SemaphoreType.DMA,
        ],
        compiler_params=pltpu.CompilerParams(
            needs_layout_passes=False, use_tc_tiling_on_sc=False),
    )
    def fm(fi, fv, em1, em2, y1, y2, idx_v, fv_v, w2_v, rows_v, y1_v, y2_v,
           sem):
        wid = lax.axis_index("s") * _NC + lax.axis_index("c")

        def chunk(c, carry):
            row0 = wid * RW + c * R
            flat0 = row0 * F
            pltpu.sync_copy(fi.at[pl.ds(flat0, NIDX)], idx_v)
            pltpu.sync_copy(fv.at[pl.ds(flat0, NIDX)], fv_v)
            handles = []
            for k in range(NSTREAM):
                sl = pl.ds(k * SLEN, SLEN)
                handles.append(
                    pltpu.async_copy(em1.at[idx_v.at[sl]], rows_v.at[sl], sem))
                handles.append(
                    pltpu.async_copy(em2.at[idx_v.at[sl]], w2_v.at[sl], sem))
            for h in handles:
                h.wait()

            for i in range(NIDX // L):
                s = pl.ds(i * L, L)
                y1_v[s] = w2_v[s] * fv_v[s]

            def row(b, _):
                fb = b * F
                z = jnp.zeros((L,), jnp.float32)
                a_lo, a_hi, s_lo, s_hi = z, z, z, z
                for f in range(F):
                    r = fb + f
                    fvf = plsc.load_gather(fv_v, [jnp.full((L,), r, jnp.int32)])
                    lo = rows_v[r, pl.ds(0, L)] * fvf
                    hi = rows_v[r, pl.ds(L, L)] * fvf
                    a_lo = a_lo + lo
                    a_hi = a_hi + hi
                    s_lo = s_lo + lo * lo
                    s_hi = s_hi + hi * hi
                y2_v[pl.ds(b * D, L)] = 0.5 * (a_lo * a_lo - s_lo)
                y2_v[pl.ds(b * D + L, L)] = 0.5 * (a_hi * a_hi - s_hi)
                return 0

            lax.fori_loop(0, R, row, 0, unroll=False)

            pltpu.sync_copy(y1_v, y1.at[pl.ds(flat0, NIDX)])
            pltpu.sync_copy(y2_v, y2.at[pl.ds(row0 * D, R * D)])
            return carry

        lax.fori_loop(0, NCHUNK, chunk, 0, unroll=False)

    return fm


def kernel(feat_index, feat_value, em1_table, em2_table):
    B, F = feat_index.shape
    V, D = em1_table.shape
    fi = feat_index.astype(jnp.int32).reshape(B * F)
    fv = feat_value.reshape(B * F)
    em2 = em2_table.reshape(V)
    em1_lin = _build_relayout(V, D)(em1_table.T).reshape(-1, D)
    y1, y2 = _build_fm(B, F, V, D, 32)(fi, fv, em1_lin, em2)
    return y1.reshape(B, F), y2.reshape(B, D)
